# baseline (speedup 1.0000x reference)
import functools

import jax
import jax.numpy as jnp
from jax import lax
from jax.experimental import pallas as pl
from jax.experimental.pallas import tpu as pltpu
from jax.experimental.pallas import tpu_sc as plsc

_NC = 2
_NS = 16
_L = 16


@functools.lru_cache(maxsize=None)
def _make_sc_kernel(B, D, V, interpret=False):
    NW = _NC * _NS
    bpw = B // NW
    nblk = bpw // _L
    nch = D // _L
    mesh = plsc.VectorSubcoreMesh(core_axis_name="c", subcore_axis_name="s")

    @functools.partial(
        pl.kernel,
        out_type=jax.ShapeDtypeStruct((B,), jnp.float32),
        mesh=mesh,
        scratch_types=[
            pltpu.VMEM((bpw,), jnp.int32),
            pltpu.VMEM((bpw, D), jnp.float32),
            pltpu.VMEM((bpw, D), jnp.float32),
            pltpu.VMEM((D,), jnp.float32),
            pltpu.VMEM((bpw,), jnp.float32),
            pltpu.SemaphoreType.DMA,
        ],
        compiler_params=pltpu.CompilerParams(
            needs_layout_passes=False, use_tc_tiling_on_sc=False),
        interpret=interpret,
    )
    def k(rain_hbm, dem_hbm, wb_hbm, table_hbm, out_hbm,
          idx_v, rows_v, rain_v, wb_v, out_v, sem):
        wid = lax.axis_index("s") * _NC + lax.axis_index("c")
        base = wid * bpw

        pltpu.sync_copy(dem_hbm.at[pl.ds(base, bpw)], idx_v)
        pltpu.sync_copy(wb_hbm, wb_v)
        pltpu.sync_copy(rain_hbm.at[pl.ds(base, bpw)], rain_v)

        for j in range(bpw // _L):
            sl = pl.ds(j * _L, _L)
            d = idx_v[sl]
            idx_v[sl] = jnp.where(d == 0, V - 1, d - 1)

        pltpu.async_copy(table_hbm.at[idx_v], rows_v, sem).wait()

        wb = [wb_v[pl.ds(c * _L, _L)] for c in range(nch)]

        lanes = lax.iota(jnp.int32, _L)
        for blk in range(nblk):
            out_block = jnp.zeros((_L,), jnp.float32)
            for i in range(_L):
                r = blk * _L + i
                acc = None
                for c in range(nch):
                    sl = pl.ds(c * _L, _L)
                    p = rain_v[r, sl] * (rows_v[r, sl] + wb[c])
                    acc = p if acc is None else acc + p
                out_block = jnp.where(lanes == i, jnp.sum(acc), out_block)
            out_v[pl.ds(blk * _L, _L)] = out_block

        pltpu.sync_copy(out_v, out_hbm.at[pl.ds(base, bpw)])

    return k


def kernel(rainfall_vec, dem_id, w_base, w_dem_delta):
    B, D = rainfall_vec.shape
    k = _make_sc_kernel(B, D, w_dem_delta.shape[0])
    y = k(rainfall_vec, dem_id.astype(jnp.int32), w_base, w_dem_delta)
    return y.reshape(B, 1)


# speedup vs baseline: 1.7188x; 1.7188x over previous
import functools

import jax
import jax.numpy as jnp
from jax import lax
from jax.experimental import pallas as pl
from jax.experimental.pallas import tpu as pltpu
from jax.experimental.pallas import tpu_sc as plsc

_NC = 2
_NS = 16
_L = 16


@functools.lru_cache(maxsize=None)
def _make_sc_kernel(B, D, V, interpret=False):
    NW = _NC * _NS
    bpw = B // NW
    nblk = bpw // _L
    nch = D // _L
    mesh = plsc.VectorSubcoreMesh(core_axis_name="c", subcore_axis_name="s")

    @functools.partial(
        pl.kernel,
        out_type=jax.ShapeDtypeStruct((B,), jnp.float32),
        mesh=mesh,
        scratch_types=[
            pltpu.VMEM((bpw,), jnp.int32),
            pltpu.VMEM((bpw, D), jnp.float32),
            pltpu.VMEM((bpw, D), jnp.float32),
            pltpu.VMEM((D,), jnp.float32),
            pltpu.VMEM((bpw,), jnp.float32),
            pltpu.SemaphoreType.DMA,
        ],
        compiler_params=pltpu.CompilerParams(needs_layout_passes=False),
        interpret=interpret,
    )
    def k(rain_hbm, dem_hbm, wb_hbm, table_hbm, out_hbm,
          idx_v, rows_v, rain_v, wb_v, out_v, sem):
        wid = lax.axis_index("s") * _NC + lax.axis_index("c")
        base = wid * bpw

        pltpu.sync_copy(dem_hbm.at[pl.ds(base, bpw)], idx_v)
        pltpu.sync_copy(wb_hbm, wb_v)
        pltpu.sync_copy(rain_hbm.at[pl.ds(base, bpw)], rain_v)

        lanes = lax.iota(jnp.int32, _L)

        copies = []
        for j in range(nblk):
            d = idx_v[pl.ds(j * _L, _L)]
            r = jnp.where(d == 0, V - 1, d - 1)
            for i in range(_L):
                r_i = jnp.sum(jnp.where(lanes == i, r, 0))
                copies.append(pltpu.async_copy(
                    table_hbm.at[r_i], rows_v.at[j * _L + i], sem))
        for cp in copies:
            cp.wait()

        wb = [wb_v[pl.ds(c * _L, _L)] for c in range(nch)]

        for blk in range(nblk):
            out_block = jnp.zeros((_L,), jnp.float32)
            for i in range(_L):
                r = blk * _L + i
                acc = None
                for c in range(nch):
                    sl = pl.ds(c * _L, _L)
                    p = rain_v[r, sl] * (rows_v[r, sl] + wb[c])
                    acc = p if acc is None else acc + p
                out_block = jnp.where(lanes == i, jnp.sum(acc), out_block)
            out_v[pl.ds(blk * _L, _L)] = out_block

        pltpu.sync_copy(out_v, out_hbm.at[pl.ds(base, bpw)])

    return k


def kernel(rainfall_vec, dem_id, w_base, w_dem_delta):
    B, D = rainfall_vec.shape
    V = w_dem_delta.shape[0]
    k = _make_sc_kernel(B, D, V)
    y = k(rainfall_vec, dem_id.astype(jnp.int32), w_base, w_dem_delta)
    return y.reshape(B, 1)


# speedup vs baseline: 14.6339x; 8.5141x over previous
import functools

import jax
import jax.numpy as jnp
from jax import lax
from jax.experimental import pallas as pl
from jax.experimental.pallas import tpu as pltpu
from jax.experimental.pallas import tpu_sc as plsc

_NC = 2
_NS = 16
_L = 16
_SW = 128
_W = 8


@functools.lru_cache(maxsize=None)
def _make_sc_kernel(B, D, V, interpret=False):
    NW = _NC * _NS
    bpw = B // NW
    nblk = bpw // _L
    nch = D // _L
    tail_start = (V // _SW) * _SW if V % _SW else V
    ntail = V - tail_start
    alt_start = tail_start - _SW
    mesh = plsc.VectorSubcoreMesh(core_axis_name="c", subcore_axis_name="s")

    @functools.partial(
        pl.kernel,
        out_type=jax.ShapeDtypeStruct((B,), jnp.float32),
        mesh=mesh,
        scratch_types=[
            pltpu.VMEM((bpw,), jnp.int32),
            pltpu.VMEM((_W, D, _SW), jnp.float32),
            pltpu.VMEM((D, _SW), jnp.float32),
            pltpu.VMEM((D, max(ntail, 1)), jnp.float32),
            pltpu.VMEM((D,), jnp.float32),
            pltpu.VMEM((bpw,), jnp.float32),
            pltpu.SemaphoreType.DMA,
        ],
        compiler_params=pltpu.CompilerParams(needs_layout_passes=False),
        interpret=interpret,
    )
    def k(rain_hbm, dem_hbm, wb_hbm, table_hbm, tail_hbm, out_hbm,
          idx_v, slab_v, rain_v, tail_v, wb_v, out_v, sem):
        wid = lax.axis_index("s") * _NC + lax.axis_index("c")
        base = wid * bpw

        pltpu.sync_copy(dem_hbm.at[pl.ds(base, bpw)], idx_v)
        pltpu.sync_copy(wb_hbm, wb_v)
        rain_blk = (base // _SW) * _SW
        rain_off = base - rain_blk
        pltpu.sync_copy(rain_hbm.at[:, pl.ds(pl.multiple_of(rain_blk, _SW),
                                             _SW)], rain_v)
        if ntail:
            pltpu.sync_copy(tail_hbm, tail_v)

        lanes = lax.iota(jnp.int32, _L)

        rs = []
        for j in range(nblk):
            d = idx_v[pl.ds(j * _L, _L)]
            rvec = jnp.where(d == 0, V - 1, d - 1)
            for i in range(_L):
                rs.append(jnp.sum(jnp.where(lanes == i, rvec, 0)))

        wb = [wb_v[pl.ds(c * _L, _L)] for c in range(nch)]
        dvecs = [c * _L + lanes for c in range(nch)]

        for blk in range(nblk):
            out_block = jnp.zeros((_L,), jnp.float32)
            for wave in range(_L // _W):
                w0 = blk * _L + wave * _W
                copies = []
                starts = []
                for w in range(_W):
                    r_i = rs[w0 + w]
                    aligned = (r_i // _SW) * _SW
                    if ntail:
                        start = jnp.where(r_i >= tail_start, alt_start,
                                          aligned)
                    else:
                        start = aligned
                    starts.append(start)
                    copies.append(pltpu.async_copy(
                        table_hbm.at[:, pl.ds(pl.multiple_of(start, _SW),
                                              _SW)],
                        slab_v.at[w], sem))
                for cp in copies:
                    cp.wait()
                for w in range(_W):
                    i = wave * _W + w
                    r_i = rs[w0 + w]
                    col = r_i - starts[w]
                    rcol = rain_off + blk * _L + i
                    if ntail:
                        is_tail = r_i >= tail_start
                        tcol = jnp.where(is_tail, r_i - tail_start, 0)
                        col = jnp.where(is_tail, 0, col)
                    acc = None
                    for c in range(nch):
                        delta_c = plsc.load_gather(
                            slab_v, [jnp.full((_L,), w, jnp.int32), dvecs[c],
                                     jnp.full((_L,), col, jnp.int32)])
                        if ntail:
                            tail_c = plsc.load_gather(
                                tail_v, [dvecs[c],
                                         jnp.full((_L,), tcol, jnp.int32)])
                            delta_c = jnp.where(is_tail, tail_c, delta_c)
                        rain_c = plsc.load_gather(
                            rain_v, [dvecs[c],
                                     jnp.full((_L,), rcol, jnp.int32)])
                        p = rain_c * (wb[c] + delta_c)
                        acc = p if acc is None else acc + p
                    out_block = jnp.where(lanes == i, jnp.sum(acc), out_block)
            out_v[pl.ds(blk * _L, _L)] = out_block

        pltpu.sync_copy(out_v, out_hbm.at[pl.ds(base, bpw)])

    return k


def kernel(rainfall_vec, dem_id, w_base, w_dem_delta):
    B, D = rainfall_vec.shape
    V = w_dem_delta.shape[0]
    tail_start = (V // _SW) * _SW if V % _SW else V
    tail = w_dem_delta[tail_start:, :].T if V % _SW else (
        jnp.zeros((D, 1), jnp.float32))
    k = _make_sc_kernel(B, D, V)
    y = k(rainfall_vec.T, dem_id.astype(jnp.int32), w_base,
          w_dem_delta.T, tail)
    return y.reshape(B, 1)
